# baseline (speedup 1.0000x reference)
import jax
import jax.numpy as jnp
from jax.experimental import pallas as pl
from jax.experimental.pallas import tpu as pltpu

_NS = 5
_BETA = 0.75
_EPS = 1e-9
_ROWS = 16384
_D = 64
_BLK = 2048


def _tc_body(idx_ref, ctx_ref, tgt_ref, emb_ref, out_ref, erows, sem):
    step = pl.program_id(0)

    @pl.when(step == 0)
    def _init():
        out_ref[0, 0] = 0.0
        erows[...] = jnp.zeros_like(erows)
        for s in range(_NS):
            cp = pltpu.make_async_copy(
                emb_ref.at[pl.ds(idx_ref[s], 1)], erows.at[pl.ds(s, 1)], sem)
            cp.start()
            cp.wait()

    c = ctx_ref[...]
    t = tgt_ref[...]
    dots = jnp.sum(c * t, axis=1, keepdims=True)
    lt = jnp.log(jnp.clip(1.0 / (1.0 + jnp.exp(-dots)), _EPS, None))
    e = erows[...]
    sdots = jax.lax.dot_general(c, e, (((1,), (1,)), ((), ())),
                                preferred_element_type=jnp.float32)
    ls = jnp.log(jnp.clip(1.0 / (1.0 + jnp.exp(sdots)), _BETA, None))
    col = jax.lax.broadcasted_iota(jnp.int32, ls.shape, 1)
    ls = jnp.where(col < _NS, ls, 0.0)
    out_ref[0, 0] += jnp.sum(lt) + jnp.sum(ls)


def kernel(context, target, emb_table, sample_indices):
    grid_spec = pltpu.PrefetchScalarGridSpec(
        num_scalar_prefetch=1,
        grid=(_ROWS // _BLK,),
        in_specs=[
            pl.BlockSpec((_BLK, _D), lambda i, idx: (i, 0)),
            pl.BlockSpec((_BLK, _D), lambda i, idx: (i, 0)),
            pl.BlockSpec(memory_space=pltpu.MemorySpace.HBM),
        ],
        out_specs=pl.BlockSpec(memory_space=pltpu.MemorySpace.SMEM),
        scratch_shapes=[
            pltpu.VMEM((8, _D), jnp.float32),
            pltpu.SemaphoreType.DMA,
        ],
    )
    out = pl.pallas_call(
        _tc_body,
        grid_spec=grid_spec,
        out_shape=jax.ShapeDtypeStruct((1, 1), jnp.float32),
    )(sample_indices.astype(jnp.int32), context, target, emb_table)
    return out[0, 0]
